# baseline (speedup 1.0000x reference)
import functools

import jax
import jax.numpy as jnp
from jax.experimental import pallas as pl
from jax.experimental.pallas import tpu as pltpu

_BM = 400


_SCALE = 4096.0


_NSPLIT = 2560


def _k2_hw(adj_ref, feat_ref, w0_ref, b0_ref, wcat_ref, bcat_ref,
           hw_ref, adj8a_ref, adj8b_ref, adj8c_ref, adj8d_ref, xw_sc):
    @pl.when(pl.program_id(0) == 0)
    def _init_xw():
        acc = jnp.dot(feat_ref[...].astype(jnp.bfloat16),
                      w0_ref[...].astype(jnp.bfloat16),
                      preferred_element_type=jnp.float32)
        xw_sc[...] = (acc + b0_ref[...]).astype(jnp.bfloat16)

    a32 = adj_ref[...]
    a = a32.astype(jnp.bfloat16)
    a8 = (a32 * _SCALE).astype(jnp.float8_e4m3fn)
    adj8a_ref[...] = a8[:, :_NSPLIT]
    adj8b_ref[...] = a8[:, _NSPLIT:2 * _NSPLIT]
    adj8c_ref[...] = a8[:, 2 * _NSPLIT:3 * _NSPLIT]
    adj8d_ref[...] = a8[:, 3 * _NSPLIT:]
    h = jnp.dot(a, xw_sc[...], preferred_element_type=jnp.float32)
    h = jnp.maximum(h, 0.0)
    hw = jnp.dot(h.astype(jnp.bfloat16), wcat_ref[...].astype(jnp.bfloat16),
                 preferred_element_type=jnp.float32) + bcat_ref[...]
    hw_ref[...] = hw.astype(jnp.float8_e4m3fn)


_BM3 = 1000
_BM4 = 200


def _k34_fused(nsteps3, adj8a_ref, adj8b_ref, adj8c_ref, adj8d_ref,
               hw_ref, noise_ref, rec_ref, mean_ref, logstd_ref, z_sc):
    i = pl.program_id(0)

    @pl.when(i < nsteps3)
    def _k3_phase():
        s = _NSPLIT
        ml = (jnp.dot(adj8a_ref[...], hw_ref[:s, :],
                      preferred_element_type=jnp.float32)
              + jnp.dot(adj8b_ref[...], hw_ref[s:2 * s, :],
                        preferred_element_type=jnp.float32)
              + jnp.dot(adj8c_ref[...], hw_ref[2 * s:3 * s, :],
                        preferred_element_type=jnp.float32)
              + jnp.dot(adj8d_ref[...], hw_ref[3 * s:, :],
                        preferred_element_type=jnp.float32)) * (1.0 / _SCALE)
        d_out = ml.shape[1] // 2
        mean = ml[:, :d_out]
        logstd = ml[:, d_out:]
        mean_ref[...] = mean
        logstd_ref[...] = logstd
        z_sc[pl.ds(i * _BM3, _BM3), :] = noise_ref[...] * jnp.exp(logstd) + mean

    @pl.when(i >= nsteps3)
    def _k4_phase():
        j = i - nsteps3
        zi = z_sc[pl.ds(j * _BM4, _BM4), :].astype(jnp.bfloat16)
        zj = z_sc[...].astype(jnp.bfloat16)
        logits = jax.lax.dot_general(zi, zj,
                                     (((1,), (1,)), ((), ())),
                                     preferred_element_type=jnp.float32)
        rec_ref[...] = 0.5 * (jnp.tanh(0.5 * logits) + 1.0)


def kernel(adj, features, W0, b0, Wm, bm, Wl, bl, noise):
    n, d_in = features.shape
    d_h = W0.shape[1]
    d_out = Wm.shape[1]
    f32 = jnp.float32

    wcat = jnp.concatenate([Wm, Wl], axis=1)
    bcat = jnp.concatenate([bm, bl])[None, :]
    b0r = b0[None, :]

    hw, adj8a, adj8b, adj8c, adj8d = pl.pallas_call(
        _k2_hw,
        grid=(n // _BM,),
        in_specs=[
            pl.BlockSpec((_BM, n), lambda i: (i, 0)),
            pl.BlockSpec((n, d_in), lambda i: (0, 0)),
            pl.BlockSpec((d_in, d_h), lambda i: (0, 0)),
            pl.BlockSpec((1, d_h), lambda i: (0, 0)),
            pl.BlockSpec((d_h, 2 * d_out), lambda i: (0, 0)),
            pl.BlockSpec((1, 2 * d_out), lambda i: (0, 0)),
        ],
        out_specs=[
            pl.BlockSpec((_BM, 2 * d_out), lambda i: (i, 0)),
            pl.BlockSpec((_BM, _NSPLIT), lambda i: (i, 0)),
            pl.BlockSpec((_BM, _NSPLIT), lambda i: (i, 0)),
            pl.BlockSpec((_BM, _NSPLIT), lambda i: (i, 0)),
            pl.BlockSpec((_BM, n - 3 * _NSPLIT), lambda i: (i, 0)),
        ],
        out_shape=[
            jax.ShapeDtypeStruct((n, 2 * d_out), jnp.float8_e4m3fn),
            jax.ShapeDtypeStruct((n, _NSPLIT), jnp.float8_e4m3fn),
            jax.ShapeDtypeStruct((n, _NSPLIT), jnp.float8_e4m3fn),
            jax.ShapeDtypeStruct((n, _NSPLIT), jnp.float8_e4m3fn),
            jax.ShapeDtypeStruct((n, n - 3 * _NSPLIT), jnp.float8_e4m3fn),
        ],
        scratch_shapes=[pltpu.VMEM((n, d_h), jnp.bfloat16)],
    )(adj, features, W0, b0r, wcat, bcat)

    ns3 = n // _BM3
    ns4 = n // _BM4
    adj_rec, mean, logstd = pl.pallas_call(
        functools.partial(_k34_fused, ns3),
        grid=(ns3 + ns4,),
        in_specs=[
            pl.BlockSpec((_BM3, _NSPLIT), lambda i: (jnp.minimum(i, ns3 - 1), 0)),
            pl.BlockSpec((_BM3, _NSPLIT), lambda i: (jnp.minimum(i, ns3 - 1), 0)),
            pl.BlockSpec((_BM3, _NSPLIT), lambda i: (jnp.minimum(i, ns3 - 1), 0)),
            pl.BlockSpec((_BM3, n - 3 * _NSPLIT), lambda i: (jnp.minimum(i, ns3 - 1), 0)),
            pl.BlockSpec((n, 2 * d_out), lambda i: (0, 0)),
            pl.BlockSpec((_BM3, d_out), lambda i: (jnp.minimum(i, ns3 - 1), 0)),
        ],
        out_specs=[
            pl.BlockSpec((_BM4, n), lambda i: (jnp.maximum(i - ns3, 0), 0)),
            pl.BlockSpec((_BM3, d_out), lambda i: (jnp.minimum(i, ns3 - 1), 0)),
            pl.BlockSpec((_BM3, d_out), lambda i: (jnp.minimum(i, ns3 - 1), 0)),
        ],
        out_shape=[
            jax.ShapeDtypeStruct((n, n), f32),
            jax.ShapeDtypeStruct((n, d_out), f32),
            jax.ShapeDtypeStruct((n, d_out), f32),
        ],
        scratch_shapes=[pltpu.VMEM((n, d_out), f32)],
    )(adj8a, adj8b, adj8c, adj8d, hw, noise)

    return (adj_rec, mean, logstd)


# speedup vs baseline: 1.0048x; 1.0048x over previous
import functools

import jax
import jax.numpy as jnp
from jax.experimental import pallas as pl
from jax.experimental.pallas import tpu as pltpu

_BM = 400


_SCALE = 4096.0


_NSPLIT = 5120


def _k2_hw(adj_ref, feat_ref, w0_ref, b0_ref, wcat_ref, bcat_ref,
           hw_ref, adj8a_ref, adj8b_ref, xw_sc):
    @pl.when(pl.program_id(0) == 0)
    def _init_xw():
        acc = jnp.dot(feat_ref[...].astype(jnp.bfloat16),
                      w0_ref[...].astype(jnp.bfloat16),
                      preferred_element_type=jnp.float32)
        xw_sc[...] = (acc + b0_ref[...]).astype(jnp.bfloat16)

    a32 = adj_ref[...]
    a = a32.astype(jnp.bfloat16)
    a8 = (a32 * _SCALE).astype(jnp.float8_e4m3fn)
    adj8a_ref[...] = a8[:, :_NSPLIT]
    adj8b_ref[...] = a8[:, _NSPLIT:]
    h = jnp.dot(a, xw_sc[...], preferred_element_type=jnp.float32)
    h = jnp.maximum(h, 0.0)
    hw = jnp.dot(h.astype(jnp.bfloat16), wcat_ref[...].astype(jnp.bfloat16),
                 preferred_element_type=jnp.float32) + bcat_ref[...]
    hw_ref[...] = hw.astype(jnp.float8_e4m3fn)


_BM3 = 1000
_BM4 = 200


def _k34_fused(nsteps3, adj8a_ref, adj8b_ref, hw_ref, noise_ref,
               rec_ref, mean_ref, logstd_ref, z_sc):
    i = pl.program_id(0)

    @pl.when(i < nsteps3)
    def _k3_phase():
        ml = (jnp.dot(adj8a_ref[...], hw_ref[:_NSPLIT, :],
                      preferred_element_type=jnp.float32)
              + jnp.dot(adj8b_ref[...], hw_ref[_NSPLIT:, :],
                        preferred_element_type=jnp.float32)) * (1.0 / _SCALE)
        d_out = ml.shape[1] // 2
        mean = ml[:, :d_out]
        logstd = ml[:, d_out:]
        mean_ref[...] = mean
        logstd_ref[...] = logstd
        z_sc[pl.ds(i * _BM3, _BM3), :] = noise_ref[...] * jnp.exp(logstd) + mean

    @pl.when(i >= nsteps3)
    def _k4_phase():
        j = i - nsteps3
        zi = z_sc[pl.ds(j * _BM4, _BM4), :].astype(jnp.bfloat16)
        zj = z_sc[...].astype(jnp.bfloat16)
        logits = jax.lax.dot_general(zi, zj,
                                     (((1,), (1,)), ((), ())),
                                     preferred_element_type=jnp.float32)
        rec_ref[...] = 0.5 * (jnp.tanh(0.5 * logits) + 1.0)


def kernel(adj, features, W0, b0, Wm, bm, Wl, bl, noise):
    n, d_in = features.shape
    d_h = W0.shape[1]
    d_out = Wm.shape[1]
    f32 = jnp.float32

    wcat = jnp.concatenate([Wm, Wl], axis=1)
    bcat = jnp.concatenate([bm, bl])[None, :]
    b0r = b0[None, :]

    hw, adj8a, adj8b = pl.pallas_call(
        _k2_hw,
        grid=(n // _BM,),
        in_specs=[
            pl.BlockSpec((_BM, n), lambda i: (i, 0)),
            pl.BlockSpec((n, d_in), lambda i: (0, 0)),
            pl.BlockSpec((d_in, d_h), lambda i: (0, 0)),
            pl.BlockSpec((1, d_h), lambda i: (0, 0)),
            pl.BlockSpec((d_h, 2 * d_out), lambda i: (0, 0)),
            pl.BlockSpec((1, 2 * d_out), lambda i: (0, 0)),
        ],
        out_specs=[
            pl.BlockSpec((_BM, 2 * d_out), lambda i: (i, 0)),
            pl.BlockSpec((_BM, _NSPLIT), lambda i: (i, 0)),
            pl.BlockSpec((_BM, n - _NSPLIT), lambda i: (i, 0)),
        ],
        out_shape=[
            jax.ShapeDtypeStruct((n, 2 * d_out), jnp.float8_e4m3fn),
            jax.ShapeDtypeStruct((n, _NSPLIT), jnp.float8_e4m3fn),
            jax.ShapeDtypeStruct((n, n - _NSPLIT), jnp.float8_e4m3fn),
        ],
        scratch_shapes=[pltpu.VMEM((n, d_h), jnp.bfloat16)],
    )(adj, features, W0, b0r, wcat, bcat)

    ns3 = n // _BM3
    ns4 = n // _BM4
    adj_rec, mean, logstd = pl.pallas_call(
        functools.partial(_k34_fused, ns3),
        grid=(ns3 + ns4,),
        in_specs=[
            pl.BlockSpec((_BM3, _NSPLIT), lambda i: (jnp.minimum(i, ns3 - 1), 0)),
            pl.BlockSpec((_BM3, n - _NSPLIT), lambda i: (jnp.minimum(i, ns3 - 1), 0)),
            pl.BlockSpec((n, 2 * d_out), lambda i: (0, 0)),
            pl.BlockSpec((_BM3, d_out), lambda i: (jnp.minimum(i, ns3 - 1), 0)),
        ],
        out_specs=[
            pl.BlockSpec((_BM4, n), lambda i: (jnp.maximum(i - ns3, 0), 0)),
            pl.BlockSpec((_BM3, d_out), lambda i: (jnp.minimum(i, ns3 - 1), 0)),
            pl.BlockSpec((_BM3, d_out), lambda i: (jnp.minimum(i, ns3 - 1), 0)),
        ],
        out_shape=[
            jax.ShapeDtypeStruct((n, n), f32),
            jax.ShapeDtypeStruct((n, d_out), f32),
            jax.ShapeDtypeStruct((n, d_out), f32),
        ],
        scratch_shapes=[pltpu.VMEM((n, d_out), f32)],
    )(adj8a, adj8b, hw, noise)

    return (adj_rec, mean, logstd)
